# baseline (speedup 1.0000x reference)
import functools

import jax
import jax.numpy as jnp
from jax import lax
from jax.experimental import pallas as pl
from jax.experimental.pallas import tpu as pltpu
from jax.experimental.pallas import tpu_sc as plsc

PATCH_K = 32
R_TILE = 64
_TOP_T = 6
_BLK = 128


def _dist_topk_body(roots_ref, points_ref, dist_out_ref,
                    idx_out_ref, kdist_out_ref):
    rr = roots_ref[0]
    pp = points_ref[0]
    n = pp.shape[0]
    nblk = n // _BLK
    inf = jnp.float32(jnp.inf)
    r0 = jnp.sum(rr * rr, axis=1)[:, None]
    r1 = jnp.sum(pp * pp, axis=1).reshape(nblk, _BLK)
    col = lax.broadcasted_iota(jnp.int32, (R_TILE, _BLK), 1)

    sv = [jnp.full((R_TILE, _BLK), inf, jnp.float32) for _ in range(_TOP_T)]
    iv = [jnp.zeros((R_TILE, _BLK), jnp.int32) for _ in range(_TOP_T)]
    for c in range(nblk):
        ppc = pp[c * _BLK:(c + 1) * _BLK, :]
        dotc = lax.dot_general(rr, ppc, (((1,), (1,)), ((), ())),
                               preferred_element_type=jnp.float32)
        dc = (r0 - 2.0 * dotc) + r1[c:c + 1, :]
        dist_out_ref[0, :, c * _BLK:(c + 1) * _BLK] = dc
        cv, ci = dc, col + (c * _BLK)
        for k in range(_TOP_T):
            swap = cv < sv[k]
            sv[k], cv = jnp.where(swap, cv, sv[k]), jnp.where(swap, sv[k], cv)
            iv[k], ci = jnp.where(swap, ci, iv[k]), jnp.where(swap, iv[k], ci)

    s_last = sv[_TOP_T - 1]

    ms, as_ = [], []
    for _ in range(PATCH_K):
        m = jnp.min(sv[0], axis=1, keepdims=True)
        a = jnp.min(jnp.where(sv[0] == m, iv[0], n), axis=1, keepdims=True)
        mask = (sv[0] == m) & (iv[0] == a)
        for k in range(_TOP_T - 1):
            sv[k] = jnp.where(mask, sv[k + 1], sv[k])
            iv[k] = jnp.where(mask, iv[k + 1], iv[k])
        sv[_TOP_T - 1] = jnp.where(mask, inf, sv[_TOP_T - 1])
        ms.append(m)
        as_.append(a)
    fast_v = jnp.concatenate(ms, axis=1)
    fast_i = jnp.concatenate(as_, axis=1)

    flag = jnp.any(s_last <= ms[-1])

    def slow(_):
        l32 = lax.broadcasted_iota(jnp.int32, (R_TILE, PATCH_K), 1)
        colf = lax.broadcasted_iota(jnp.int32, (R_TILE, n), 1)

        def step(j, carry):
            out_v, out_i, m_prev, a_prev = carry
            dj = dist_out_ref[0]
            live = (dj > m_prev) | ((dj == m_prev) & (colf > a_prev))
            dj = jnp.where(live, dj, inf)
            m = jnp.min(dj, axis=1, keepdims=True)
            a = jnp.min(jnp.where(dj == m, colf, n), axis=1, keepdims=True)
            out_v = jnp.where(l32 == j, m, out_v)
            out_i = jnp.where(l32 == j, a, out_i)
            return out_v, out_i, m, a

        out_v = jnp.zeros((R_TILE, PATCH_K), jnp.float32)
        out_i = jnp.zeros((R_TILE, PATCH_K), jnp.int32)
        m0 = jnp.full((R_TILE, 1), -inf, jnp.float32)
        a0 = jnp.full((R_TILE, 1), -1, jnp.int32)
        out_v, out_i, _, _ = lax.fori_loop(0, PATCH_K, step,
                                           (out_v, out_i, m0, a0))
        return out_v, out_i

    out_v, out_i = lax.cond(flag, slow, lambda _: (fast_v, fast_i), None)
    idx_out_ref[0] = out_i
    kdist_out_ref[0] = jnp.sqrt(jnp.maximum(out_v, 1e-5))


@functools.partial(jax.jit, static_argnums=())
def _dist_topk(points, roots):
    B, N, _ = points.shape
    R = roots.shape[1]
    grid = (B, R // R_TILE)
    return pl.pallas_call(
        _dist_topk_body,
        grid=grid,
        in_specs=[
            pl.BlockSpec((1, R_TILE, 3), lambda b, r: (b, r, 0)),
            pl.BlockSpec((1, N, 3), lambda b, r: (b, 0, 0)),
        ],
        out_specs=[
            pl.BlockSpec((1, R_TILE, N), lambda b, r: (b, r, 0)),
            pl.BlockSpec((1, R_TILE, PATCH_K), lambda b, r: (b, r, 0)),
            pl.BlockSpec((1, R_TILE, PATCH_K), lambda b, r: (b, r, 0)),
        ],
        out_shape=[
            jax.ShapeDtypeStruct((B, R, N), jnp.float32),
            jax.ShapeDtypeStruct((B, R, PATCH_K), jnp.int32),
            jax.ShapeDtypeStruct((B, R, PATCH_K), jnp.float32),
        ],
        compiler_params=pltpu.CompilerParams(
            dimension_semantics=("parallel", "parallel")),
    )(roots, points)


_GATHER_G = 128


def _make_sc_gather(B, N, R):
    info = plsc.get_sparse_core_info()
    nw = info.num_cores * info.num_subcores
    total = B * R * PATCH_K
    per_w = total // nw
    assert per_w % _GATHER_G == 0 and (R * PATCH_K) % per_w == 0
    nroots_w = per_w // PATCH_K
    nin = per_w // _GATHER_G
    mesh = plsc.VectorSubcoreMesh(core_axis_name="c", subcore_axis_name="s")

    import functools as _ft

    @_ft.partial(
        pl.kernel, mesh=mesh,
        out_type=jax.ShapeDtypeStruct((total * 8,), jnp.float32),
        scratch_types=[
            pltpu.VMEM((N * 8,), jnp.float32),
            pltpu.VMEM((_GATHER_G,), jnp.int32),
            pltpu.VMEM((_GATHER_G * 8,), jnp.float32),
            pltpu.VMEM((nroots_w * 16,), jnp.float32),
        ],
        compiler_params=pltpu.CompilerParams(needs_layout_passes=False),
    )
    def sc_gather(points8_hbm, idx_hbm, negroots_hbm, out_hbm,
                  pts_v, idx_v, rows_v, negr_v):
        wid = lax.axis_index("s") * info.num_cores + lax.axis_index("c")
        base_rows = wid * per_w
        b = base_rows // (R * PATCH_K)
        pbase = pl.multiple_of(b * N * 8, N * 8)
        pltpu.sync_copy(points8_hbm.at[pl.ds(pbase, N * 8)], pts_v)
        rootbase = pl.multiple_of((base_rows // PATCH_K) * 16, nroots_w * 16)
        pltpu.sync_copy(negroots_hbm.at[pl.ds(rootbase, nroots_w * 16)], negr_v)

        lane = lax.iota(jnp.int32, 16)
        lrow = lane // 8
        lcol = lane % 8

        def body(it, carry):
            base = pl.multiple_of(base_rows + it * _GATHER_G, _GATHER_G)
            pltpu.sync_copy(idx_hbm.at[pl.ds(base, _GATHER_G)], idx_v)
            for chunk in range(_GATHER_G * 8 // 16):
                rowsel = lrow + (2 * chunk)
                prow = plsc.load_gather(idx_v, [rowsel])
                vals = plsc.load_gather(pts_v, [prow * 8 + lcol])
                rl = it * (_GATHER_G // PATCH_K) + (chunk // (PATCH_K // 2))
                rv = plsc.load_gather(negr_v, [lane + rl * 16])
                rows_v[pl.ds(chunk * 16, 16)] = vals + rv
            pltpu.sync_copy(rows_v, out_hbm.at[pl.ds(base * 8, _GATHER_G * 8)])
            return carry

        lax.fori_loop(0, nin, body, jnp.int32(0))

    return sc_gather


def kernel(points, roots):
    B, N, _ = points.shape
    R = roots.shape[1]
    sq_distance_mat, knn_idx, patches_dist = _dist_topk(points, roots)

    batch_idx = jnp.broadcast_to(
        jnp.arange(B, dtype=knn_idx.dtype).reshape(B, 1, 1), (B, R, PATCH_K))
    patches_idx = jnp.stack([batch_idx, knn_idx], axis=-1)

    table = jnp.pad(points.reshape(B * N, 3), ((0, 0), (0, 5))).reshape(-1)
    negroots = jnp.tile(jnp.pad(-roots.reshape(B * R, 3), ((0, 0), (0, 5))),
                        (1, 2)).reshape(-1)
    patches8 = _make_sc_gather(B, N, R)(
        table, knn_idx.reshape(-1), negroots)
    patches = patches8.reshape(B * R * PATCH_K, 8)[:, :3].reshape(
        B, R, PATCH_K, 3)
    return (patches, patches_idx, patches_dist, sq_distance_mat)


# speedup vs baseline: 1.2704x; 1.2704x over previous
import functools

import jax
import jax.numpy as jnp
from jax import lax
from jax.experimental import pallas as pl
from jax.experimental.pallas import tpu as pltpu
from jax.experimental.pallas import tpu_sc as plsc

PATCH_K = 32
R_TILE = 128
_TOP_T = 6
_BLK = 128


def _dist_topk_body(roots_ref, points_ref, dist_out_ref,
                    idx_out_ref, kdist_out_ref):
    rr = roots_ref[0]
    pp = points_ref[0]
    n = pp.shape[0]
    nblk = n // _BLK
    inf = jnp.float32(jnp.inf)
    r0 = jnp.sum(rr * rr, axis=1)[:, None]
    r1 = jnp.sum(pp * pp, axis=1).reshape(nblk, _BLK)
    col = lax.broadcasted_iota(jnp.int32, (R_TILE, _BLK), 1)

    sv = [jnp.full((R_TILE, _BLK), inf, jnp.float32) for _ in range(_TOP_T)]
    iv = [jnp.zeros((R_TILE, _BLK), jnp.int32) for _ in range(_TOP_T)]
    for c in range(nblk):
        ppc = pp[c * _BLK:(c + 1) * _BLK, :]
        dotc = lax.dot_general(rr, ppc, (((1,), (1,)), ((), ())),
                               preferred_element_type=jnp.float32)
        dc = (r0 - 2.0 * dotc) + r1[c:c + 1, :]
        dist_out_ref[0, :, c * _BLK:(c + 1) * _BLK] = dc
        cv, ci = dc, col + (c * _BLK)
        for k in range(_TOP_T):
            swap = cv < sv[k]
            sv[k], cv = jnp.where(swap, cv, sv[k]), jnp.where(swap, sv[k], cv)
            iv[k], ci = jnp.where(swap, ci, iv[k]), jnp.where(swap, iv[k], ci)

    s_last = sv[_TOP_T - 1]

    ms, as_ = [], []
    for _ in range(PATCH_K):
        m = jnp.min(sv[0], axis=1, keepdims=True)
        a = jnp.min(jnp.where(sv[0] == m, iv[0], n), axis=1, keepdims=True)
        mask = (sv[0] == m) & (iv[0] == a)
        for k in range(_TOP_T - 1):
            sv[k] = jnp.where(mask, sv[k + 1], sv[k])
            iv[k] = jnp.where(mask, iv[k + 1], iv[k])
        sv[_TOP_T - 1] = jnp.where(mask, inf, sv[_TOP_T - 1])
        ms.append(m)
        as_.append(a)
    fast_v = jnp.concatenate(ms, axis=1)
    fast_i = jnp.concatenate(as_, axis=1)

    flag = jnp.any(s_last <= ms[-1])

    def slow(_):
        l32 = lax.broadcasted_iota(jnp.int32, (R_TILE, PATCH_K), 1)
        colf = lax.broadcasted_iota(jnp.int32, (R_TILE, n), 1)

        def step(j, carry):
            out_v, out_i, m_prev, a_prev = carry
            dj = dist_out_ref[0]
            live = (dj > m_prev) | ((dj == m_prev) & (colf > a_prev))
            dj = jnp.where(live, dj, inf)
            m = jnp.min(dj, axis=1, keepdims=True)
            a = jnp.min(jnp.where(dj == m, colf, n), axis=1, keepdims=True)
            out_v = jnp.where(l32 == j, m, out_v)
            out_i = jnp.where(l32 == j, a, out_i)
            return out_v, out_i, m, a

        out_v = jnp.zeros((R_TILE, PATCH_K), jnp.float32)
        out_i = jnp.zeros((R_TILE, PATCH_K), jnp.int32)
        m0 = jnp.full((R_TILE, 1), -inf, jnp.float32)
        a0 = jnp.full((R_TILE, 1), -1, jnp.int32)
        out_v, out_i, _, _ = lax.fori_loop(0, PATCH_K, step,
                                           (out_v, out_i, m0, a0))
        return out_v, out_i

    out_v, out_i = lax.cond(flag, slow, lambda _: (fast_v, fast_i), None)
    idx_out_ref[0] = out_i
    kdist_out_ref[0] = jnp.sqrt(jnp.maximum(out_v, 1e-5))


@functools.partial(jax.jit, static_argnums=())
def _dist_topk(points, roots):
    B, N, _ = points.shape
    R = roots.shape[1]
    grid = (B, R // R_TILE)
    return pl.pallas_call(
        _dist_topk_body,
        grid=grid,
        in_specs=[
            pl.BlockSpec((1, R_TILE, 3), lambda b, r: (b, r, 0)),
            pl.BlockSpec((1, N, 3), lambda b, r: (b, 0, 0)),
        ],
        out_specs=[
            pl.BlockSpec((1, R_TILE, N), lambda b, r: (b, r, 0)),
            pl.BlockSpec((1, R_TILE, PATCH_K), lambda b, r: (b, r, 0)),
            pl.BlockSpec((1, R_TILE, PATCH_K), lambda b, r: (b, r, 0)),
        ],
        out_shape=[
            jax.ShapeDtypeStruct((B, R, N), jnp.float32),
            jax.ShapeDtypeStruct((B, R, PATCH_K), jnp.int32),
            jax.ShapeDtypeStruct((B, R, PATCH_K), jnp.float32),
        ],
        compiler_params=pltpu.CompilerParams(
            dimension_semantics=("parallel", "parallel")),
    )(roots, points)


_GATHER_G = 128


def _make_sc_gather(B, N, R):
    info = plsc.get_sparse_core_info()
    nw = info.num_cores * info.num_subcores
    total = B * R * PATCH_K
    per_w = total // nw
    assert per_w % _GATHER_G == 0 and (R * PATCH_K) % per_w == 0
    nroots_w = per_w // PATCH_K
    nin = per_w // _GATHER_G
    mesh = plsc.VectorSubcoreMesh(core_axis_name="c", subcore_axis_name="s")

    import functools as _ft

    @_ft.partial(
        pl.kernel, mesh=mesh,
        out_type=jax.ShapeDtypeStruct((total * 8,), jnp.float32),
        scratch_types=[
            pltpu.VMEM((N * 8,), jnp.float32),
            pltpu.VMEM((_GATHER_G,), jnp.int32),
            pltpu.VMEM((_GATHER_G * 8,), jnp.float32),
            pltpu.VMEM((nroots_w * 16,), jnp.float32),
        ],
        compiler_params=pltpu.CompilerParams(needs_layout_passes=False),
    )
    def sc_gather(points8_hbm, idx_hbm, negroots_hbm, out_hbm,
                  pts_v, idx_v, rows_v, negr_v):
        wid = lax.axis_index("s") * info.num_cores + lax.axis_index("c")
        base_rows = wid * per_w
        b = base_rows // (R * PATCH_K)
        pbase = pl.multiple_of(b * N * 8, N * 8)
        pltpu.sync_copy(points8_hbm.at[pl.ds(pbase, N * 8)], pts_v)
        rootbase = pl.multiple_of((base_rows // PATCH_K) * 16, nroots_w * 16)
        pltpu.sync_copy(negroots_hbm.at[pl.ds(rootbase, nroots_w * 16)], negr_v)

        lane = lax.iota(jnp.int32, 16)
        lrow = lane // 8
        lcol = lane % 8

        def body(it, carry):
            base = pl.multiple_of(base_rows + it * _GATHER_G, _GATHER_G)
            pltpu.sync_copy(idx_hbm.at[pl.ds(base, _GATHER_G)], idx_v)
            for chunk in range(_GATHER_G * 8 // 16):
                rowsel = lrow + (2 * chunk)
                prow = plsc.load_gather(idx_v, [rowsel])
                vals = plsc.load_gather(pts_v, [prow * 8 + lcol])
                rl = it * (_GATHER_G // PATCH_K) + (chunk // (PATCH_K // 2))
                rv = plsc.load_gather(negr_v, [lane + rl * 16])
                rows_v[pl.ds(chunk * 16, 16)] = vals + rv
            pltpu.sync_copy(rows_v, out_hbm.at[pl.ds(base * 8, _GATHER_G * 8)])
            return carry

        lax.fori_loop(0, nin, body, jnp.int32(0))

    return sc_gather


def kernel(points, roots):
    B, N, _ = points.shape
    R = roots.shape[1]
    sq_distance_mat, knn_idx, patches_dist = _dist_topk(points, roots)

    batch_idx = jnp.broadcast_to(
        jnp.arange(B, dtype=knn_idx.dtype).reshape(B, 1, 1), (B, R, PATCH_K))
    patches_idx = jnp.stack([batch_idx, knn_idx], axis=-1)

    table = jnp.pad(points.reshape(B * N, 3), ((0, 0), (0, 5))).reshape(-1)
    negroots = jnp.tile(jnp.pad(-roots.reshape(B * R, 3), ((0, 0), (0, 5))),
                        (1, 2)).reshape(-1)
    patches8 = _make_sc_gather(B, N, R)(
        table, knn_idx.reshape(-1), negroots)
    patches = patches8.reshape(B * R * PATCH_K, 8)[:, :3].reshape(
        B, R, PATCH_K, 3)
    return (patches, patches_idx, patches_dist, sq_distance_mat)


# speedup vs baseline: 1.3254x; 1.0433x over previous
import functools

import jax
import jax.numpy as jnp
from jax import lax
from jax.experimental import pallas as pl
from jax.experimental.pallas import tpu as pltpu
from jax.experimental.pallas import tpu_sc as plsc

PATCH_K = 32
R_TILE = 256
_TOP_T = 6
_BLK = 128


def _dist_topk_body(roots_ref, points_ref, dist_out_ref,
                    idx_out_ref, kdist_out_ref):
    rr = roots_ref[0]
    pp = points_ref[0]
    n = pp.shape[0]
    nblk = n // _BLK
    inf = jnp.float32(jnp.inf)
    r0 = jnp.sum(rr * rr, axis=1)[:, None]
    r1 = jnp.sum(pp * pp, axis=1).reshape(nblk, _BLK)
    col = lax.broadcasted_iota(jnp.int32, (R_TILE, _BLK), 1)

    sv = [jnp.full((R_TILE, _BLK), inf, jnp.float32) for _ in range(_TOP_T)]
    iv = [jnp.zeros((R_TILE, _BLK), jnp.int32) for _ in range(_TOP_T)]
    for c in range(nblk):
        ppc = pp[c * _BLK:(c + 1) * _BLK, :]
        dotc = lax.dot_general(rr, ppc, (((1,), (1,)), ((), ())),
                               preferred_element_type=jnp.float32)
        dc = (r0 - 2.0 * dotc) + r1[c:c + 1, :]
        dist_out_ref[0, :, c * _BLK:(c + 1) * _BLK] = dc
        cv, ci = dc, col + (c * _BLK)
        for k in range(_TOP_T):
            swap = cv < sv[k]
            sv[k], cv = jnp.where(swap, cv, sv[k]), jnp.where(swap, sv[k], cv)
            iv[k], ci = jnp.where(swap, ci, iv[k]), jnp.where(swap, iv[k], ci)

    s_last = sv[_TOP_T - 1]

    ms, as_ = [], []
    for _ in range(PATCH_K):
        m = jnp.min(sv[0], axis=1, keepdims=True)
        a = jnp.min(jnp.where(sv[0] == m, iv[0], n), axis=1, keepdims=True)
        mask = (sv[0] == m) & (iv[0] == a)
        for k in range(_TOP_T - 1):
            sv[k] = jnp.where(mask, sv[k + 1], sv[k])
            iv[k] = jnp.where(mask, iv[k + 1], iv[k])
        sv[_TOP_T - 1] = jnp.where(mask, inf, sv[_TOP_T - 1])
        ms.append(m)
        as_.append(a)
    fast_v = jnp.concatenate(ms, axis=1)
    fast_i = jnp.concatenate(as_, axis=1)

    flag = jnp.any(s_last <= ms[-1])

    def slow(_):
        l32 = lax.broadcasted_iota(jnp.int32, (R_TILE, PATCH_K), 1)
        colf = lax.broadcasted_iota(jnp.int32, (R_TILE, n), 1)

        def step(j, carry):
            out_v, out_i, m_prev, a_prev = carry
            dj = dist_out_ref[0]
            live = (dj > m_prev) | ((dj == m_prev) & (colf > a_prev))
            dj = jnp.where(live, dj, inf)
            m = jnp.min(dj, axis=1, keepdims=True)
            a = jnp.min(jnp.where(dj == m, colf, n), axis=1, keepdims=True)
            out_v = jnp.where(l32 == j, m, out_v)
            out_i = jnp.where(l32 == j, a, out_i)
            return out_v, out_i, m, a

        out_v = jnp.zeros((R_TILE, PATCH_K), jnp.float32)
        out_i = jnp.zeros((R_TILE, PATCH_K), jnp.int32)
        m0 = jnp.full((R_TILE, 1), -inf, jnp.float32)
        a0 = jnp.full((R_TILE, 1), -1, jnp.int32)
        out_v, out_i, _, _ = lax.fori_loop(0, PATCH_K, step,
                                           (out_v, out_i, m0, a0))
        return out_v, out_i

    out_v, out_i = lax.cond(flag, slow, lambda _: (fast_v, fast_i), None)
    idx_out_ref[0] = out_i
    kdist_out_ref[0] = jnp.sqrt(jnp.maximum(out_v, 1e-5))


@functools.partial(jax.jit, static_argnums=())
def _dist_topk(points, roots):
    B, N, _ = points.shape
    R = roots.shape[1]
    grid = (B, R // R_TILE)
    return pl.pallas_call(
        _dist_topk_body,
        grid=grid,
        in_specs=[
            pl.BlockSpec((1, R_TILE, 3), lambda b, r: (b, r, 0)),
            pl.BlockSpec((1, N, 3), lambda b, r: (b, 0, 0)),
        ],
        out_specs=[
            pl.BlockSpec((1, R_TILE, N), lambda b, r: (b, r, 0)),
            pl.BlockSpec((1, R_TILE, PATCH_K), lambda b, r: (b, r, 0)),
            pl.BlockSpec((1, R_TILE, PATCH_K), lambda b, r: (b, r, 0)),
        ],
        out_shape=[
            jax.ShapeDtypeStruct((B, R, N), jnp.float32),
            jax.ShapeDtypeStruct((B, R, PATCH_K), jnp.int32),
            jax.ShapeDtypeStruct((B, R, PATCH_K), jnp.float32),
        ],
        compiler_params=pltpu.CompilerParams(
            dimension_semantics=("parallel", "parallel")),
    )(roots, points)


_GATHER_G = 128


def _make_sc_gather(B, N, R):
    info = plsc.get_sparse_core_info()
    nw = info.num_cores * info.num_subcores
    total = B * R * PATCH_K
    per_w = total // nw
    assert per_w % _GATHER_G == 0 and (R * PATCH_K) % per_w == 0
    nroots_w = per_w // PATCH_K
    nin = per_w // _GATHER_G
    mesh = plsc.VectorSubcoreMesh(core_axis_name="c", subcore_axis_name="s")

    import functools as _ft

    @_ft.partial(
        pl.kernel, mesh=mesh,
        out_type=jax.ShapeDtypeStruct((total * 8,), jnp.float32),
        scratch_types=[
            pltpu.VMEM((N * 8,), jnp.float32),
            pltpu.VMEM((_GATHER_G,), jnp.int32),
            pltpu.VMEM((_GATHER_G * 8,), jnp.float32),
            pltpu.VMEM((nroots_w * 16,), jnp.float32),
        ],
        compiler_params=pltpu.CompilerParams(needs_layout_passes=False),
    )
    def sc_gather(points8_hbm, idx_hbm, negroots_hbm, out_hbm,
                  pts_v, idx_v, rows_v, negr_v):
        wid = lax.axis_index("s") * info.num_cores + lax.axis_index("c")
        base_rows = wid * per_w
        b = base_rows // (R * PATCH_K)
        pbase = pl.multiple_of(b * N * 8, N * 8)
        pltpu.sync_copy(points8_hbm.at[pl.ds(pbase, N * 8)], pts_v)
        rootbase = pl.multiple_of((base_rows // PATCH_K) * 16, nroots_w * 16)
        pltpu.sync_copy(negroots_hbm.at[pl.ds(rootbase, nroots_w * 16)], negr_v)

        lane = lax.iota(jnp.int32, 16)
        lrow = lane // 8
        lcol = lane % 8

        def body(it, carry):
            base = pl.multiple_of(base_rows + it * _GATHER_G, _GATHER_G)
            pltpu.sync_copy(idx_hbm.at[pl.ds(base, _GATHER_G)], idx_v)
            for chunk in range(_GATHER_G * 8 // 16):
                rowsel = lrow + (2 * chunk)
                prow = plsc.load_gather(idx_v, [rowsel])
                vals = plsc.load_gather(pts_v, [prow * 8 + lcol])
                rl = it * (_GATHER_G // PATCH_K) + (chunk // (PATCH_K // 2))
                rv = plsc.load_gather(negr_v, [lane + rl * 16])
                rows_v[pl.ds(chunk * 16, 16)] = vals + rv
            pltpu.sync_copy(rows_v, out_hbm.at[pl.ds(base * 8, _GATHER_G * 8)])
            return carry

        lax.fori_loop(0, nin, body, jnp.int32(0))

    return sc_gather


def kernel(points, roots):
    B, N, _ = points.shape
    R = roots.shape[1]
    sq_distance_mat, knn_idx, patches_dist = _dist_topk(points, roots)

    batch_idx = jnp.broadcast_to(
        jnp.arange(B, dtype=knn_idx.dtype).reshape(B, 1, 1), (B, R, PATCH_K))
    patches_idx = jnp.stack([batch_idx, knn_idx], axis=-1)

    table = jnp.pad(points.reshape(B * N, 3), ((0, 0), (0, 5))).reshape(-1)
    negroots = jnp.tile(jnp.pad(-roots.reshape(B * R, 3), ((0, 0), (0, 5))),
                        (1, 2)).reshape(-1)
    patches8 = _make_sc_gather(B, N, R)(
        table, knn_idx.reshape(-1), negroots)
    patches = patches8.reshape(B * R * PATCH_K, 8)[:, :3].reshape(
        B, R, PATCH_K, 3)
    return (patches, patches_idx, patches_dist, sq_distance_mat)
